# baseline (speedup 1.0000x reference)
import functools

import jax
import jax.numpy as jnp
from jax import lax
from jax.experimental import pallas as pl
from jax.experimental.pallas import tpu as pltpu
from jax.experimental.pallas import tpu_sc as plsc

N = 10000
E = 320000
NCORE = 2
NSUB = 16
NW = NCORE * NSUB
CH = 128
NCHUNK = 160
EP = NSUB * NCHUNK * CH
NP = 10112
EPW = E // NW
NSLOT = 5
NGAHEAD = 2
ZROWS = NP // NSUB
OROWS = 624
OTAIL = N - OROWS * NSUB

_mesh = plsc.VectorSubcoreMesh(core_axis_name="c", subcore_axis_name="s")
_params = pltpu.CompilerParams(use_tc_tiling_on_sc=False)
_params_head = pltpu.CompilerParams(use_tc_tiling_on_sc=False,
                                    needs_layout_passes=False)


def _stage_and_zero(srcp_hbm, dstp_hbm, z_hbm, srcv, dstv, aggsh, sid):
    pltpu.sync_copy(srcp_hbm.at[sid], srcv)
    pltpu.sync_copy(dstp_hbm.at[sid], dstv)
    pltpu.sync_copy(z_hbm.at[pl.ds(sid * ZROWS, ZROWS)],
                    aggsh.at[pl.ds(sid * ZROWS, ZROWS)])


def _copy_out(aggsh, out_hbm, cid, sid):
    pltpu.sync_copy(aggsh.at[pl.ds(sid * OROWS, OROWS)],
                    out_hbm.at[cid, pl.ds(sid * OROWS, OROWS)])

    @pl.when(sid == 0)
    def _():
        pltpu.sync_copy(aggsh.at[pl.ds(OROWS * NSUB, OTAIL)],
                        out_hbm.at[cid, pl.ds(OROWS * NSUB, OTAIL)])


def _gather_scatter_loop(src_hbm, srcv, dstv, bufs, gsems, ssems, aggsh):
    for k in range(NGAHEAD):
        pltpu.async_copy(src_hbm.at[srcv.at[k]], bufs[k], gsems[k])

    def body(t, carry):
        base = t * NSLOT
        for k in range(NSLOT):
            j = base + k
            pltpu.make_async_copy(src_hbm.at[srcv.at[j]],
                                  bufs[k], gsems[k]).wait()
            pltpu.async_copy(bufs[k], aggsh.at[dstv.at[j]], ssems[k],
                             add=True)
            f = j + NGAHEAD
            kf = (k + NGAHEAD) % NSLOT

            @pl.when(f >= NSLOT)
            def _():
                pltpu.make_async_copy(bufs[kf], aggsh.at[dstv.at[0]],
                                      ssems[kf]).wait()

            @pl.when(f < NCHUNK)
            def _():
                pltpu.async_copy(src_hbm.at[srcv.at[f]], bufs[kf],
                                 gsems[kf])
        return carry

    lax.fori_loop(0, NCHUNK // NSLOT, body, 0)
    for j in range(NCHUNK - NSLOT + NGAHEAD, NCHUNK):
        k = j % NSLOT
        pltpu.make_async_copy(bufs[k], aggsh.at[dstv.at[0]],
                              ssems[k]).wait()


@functools.partial(
    pl.kernel,
    out_type=jax.ShapeDtypeStruct((NCORE, N, 16), jnp.float32),
    mesh=_mesh,
    scratch_types=[
        pltpu.VMEM((NCHUNK, CH), jnp.int32),
        pltpu.VMEM((NCHUNK, CH), jnp.int32),
    ]
    + [pltpu.VMEM((CH, 16), jnp.float32) for _ in range(NSLOT)]
    + [pltpu.VMEM_SHARED((NP, 16), jnp.float32)]
    + [pltpu.SemaphoreType.DMA for _ in range(2 * NSLOT)],
    compiler_params=_params,
)
def _sc_conv1(h0_hbm, srcp_hbm, dstp_hbm, z_hbm, out_hbm,
              srcv, dstv, *rest):
    bufs = rest[:NSLOT]
    aggsh = rest[NSLOT]
    gsems = rest[NSLOT + 1:2 * NSLOT + 1]
    ssems = rest[2 * NSLOT + 1:]
    cid = lax.axis_index("c")
    sid = lax.axis_index("s")
    _stage_and_zero(srcp_hbm, dstp_hbm, z_hbm, srcv, dstv, aggsh, sid)
    plsc.subcore_barrier()

    @pl.when(cid == 0)
    def _():
        _gather_scatter_loop(h0_hbm, srcv, dstv, bufs, gsems, ssems, aggsh)

    @pl.when(cid == 1)
    def _():
        ones_col = jnp.where(lax.iota(jnp.int32, 16) == 0, 1.0, 0.0)

        def fill(r, carry):
            bufs[0][r] = ones_col
            return carry

        lax.fori_loop(0, CH, fill, 0)

        for k in range(NSLOT):
            pltpu.async_copy(bufs[0], aggsh.at[dstv.at[k]], ssems[k],
                             add=True)

        def body(t, carry):
            base = t * NSLOT
            for k in range(NSLOT):
                j = base + k
                pltpu.make_async_copy(bufs[0], aggsh.at[dstv.at[0]],
                                      ssems[k]).wait()
                nxt = j + NSLOT

                @pl.when(nxt < NCHUNK)
                def _():
                    pltpu.async_copy(bufs[0], aggsh.at[dstv.at[nxt]],
                                     ssems[k], add=True)
            return carry

        lax.fori_loop(0, NCHUNK // NSLOT, body, 0)

    plsc.subcore_barrier()
    _copy_out(aggsh, out_hbm, cid, sid)


@functools.partial(
    pl.kernel,
    out_type=jax.ShapeDtypeStruct((NCORE, N, 64), jnp.float32),
    mesh=_mesh,
    scratch_types=[
        pltpu.VMEM((NCHUNK, CH), jnp.int32),
        pltpu.VMEM((NCHUNK, CH), jnp.int32),
    ]
    + [pltpu.VMEM((CH, 64), jnp.float32) for _ in range(NSLOT)]
    + [pltpu.VMEM_SHARED((NP, 64), jnp.float32)]
    + [pltpu.SemaphoreType.DMA for _ in range(2 * NSLOT)],
    compiler_params=_params,
)
def _sc_conv2(h1p_hbm, srcp_hbm, dstp_hbm, z_hbm, out_hbm,
              srcv, dstv, *rest):
    bufs = rest[:NSLOT]
    aggsh = rest[NSLOT]
    gsems = rest[NSLOT + 1:2 * NSLOT + 1]
    ssems = rest[2 * NSLOT + 1:]
    cid = lax.axis_index("c")
    sid = lax.axis_index("s")
    _stage_and_zero(srcp_hbm, dstp_hbm, z_hbm, srcv, dstv, aggsh, sid)
    plsc.subcore_barrier()
    _gather_scatter_loop(h1p_hbm.at[cid], srcv, dstv, bufs, gsems, ssems,
                         aggsh)
    plsc.subcore_barrier()
    _copy_out(aggsh, out_hbm, cid, sid)


@functools.partial(
    pl.kernel,
    out_type=(jax.ShapeDtypeStruct((NW, EPW), jnp.float32),
              jax.ShapeDtypeStruct((NW, EPW), jnp.float32)),
    mesh=_mesh,
    scratch_types=[
        pltpu.VMEM((EPW,), jnp.int32),
        pltpu.VMEM((EPW,), jnp.int32),
        pltpu.VMEM((N,), jnp.float32),
        pltpu.VMEM((N,), jnp.float32),
        pltpu.VMEM((EPW,), jnp.float32),
        pltpu.VMEM((EPW,), jnp.float32),
    ],
    compiler_params=_params_head,
)
def _sc_head(sl_hbm, sr_hbm, src_hbm, dst_hbm, raw_hbm, sig_hbm,
             srcv, dstv, slv, srv, rawv, sigv):
    cid = lax.axis_index("c")
    sid = lax.axis_index("s")
    wid = sid * NCORE + cid
    pltpu.sync_copy(src_hbm.at[wid], srcv)
    pltpu.sync_copy(dst_hbm.at[wid], dstv)
    pltpu.sync_copy(sl_hbm, slv)
    pltpu.sync_copy(sr_hbm, srv)

    def body(i, carry):
        o = i * 16
        s = srcv[pl.ds(o, 16)]
        d = dstv[pl.ds(o, 16)]
        raw = plsc.load_gather(slv, [s]) + plsc.load_gather(srv, [d])
        rawv[pl.ds(o, 16)] = raw
        sigv[pl.ds(o, 16)] = 1.0 / (1.0 + jnp.exp(-raw))
        return carry

    lax.fori_loop(0, EPW // 16, body, 0)
    pltpu.sync_copy(rawv, raw_hbm.at[wid])
    pltpu.sync_copy(sigv, sig_hbm.at[wid])


def _tc_lin0_body(x_ref, w_ref, b_ref, o_ref):
    o_ref[...] = (jnp.dot(x_ref[...], w_ref[...],
                          preferred_element_type=jnp.float32)
                  + b_ref[...][None, :])


def _tc_mid_body(p_ref, h0_ref, wl_ref, wr_ref, b_ref, h1p_ref, cnt_ref):
    cnt = jnp.maximum(p_ref[1][:, :1], 1.0)
    mean = p_ref[0] / cnt
    h1 = (jnp.dot(mean, wl_ref[...], preferred_element_type=jnp.float32)
          + jnp.dot(h0_ref[...], wr_ref[...],
                    preferred_element_type=jnp.float32)
          + b_ref[...][None, :])
    h1 = jnp.maximum(h1, 0.0)
    h1p_ref[0] = h1[:, :64]
    h1p_ref[1] = h1[:, 64:]
    cnt_ref[...] = cnt


def _tc_out_body(p_ref, cnt_ref, h1p_ref, wl_ref, wr_ref, b_ref,
                 wpl_ref, wpr_ref, bp_ref, sl_ref, sr_ref):
    cnt = cnt_ref[...]
    h2 = (jnp.dot(p_ref[0] / cnt, wl_ref[...][:64],
                  preferred_element_type=jnp.float32)
          + jnp.dot(p_ref[1] / cnt, wl_ref[...][64:],
                    preferred_element_type=jnp.float32)
          + jnp.dot(h1p_ref[0], wr_ref[...][:64],
                    preferred_element_type=jnp.float32)
          + jnp.dot(h1p_ref[1], wr_ref[...][64:],
                    preferred_element_type=jnp.float32)
          + b_ref[...][None, :])
    sl_ref[...] = (jnp.dot(h2, wpl_ref[...],
                           preferred_element_type=jnp.float32)
                   + bp_ref[...][None, :])
    sr_ref[...] = jnp.dot(h2, wpr_ref[...],
                          preferred_element_type=jnp.float32)


def kernel(x, edge_index, edge_attr, W0, b0, Wl1, Wr1, b1,
           Wl2, Wr2, b2, Wp, bp):
    f32 = jnp.float32
    src = edge_index[0]
    dst = edge_index[1]

    h0 = pl.pallas_call(
        _tc_lin0_body,
        out_shape=jax.ShapeDtypeStruct((N, 16), f32),
    )(x, W0.T, b0)

    pad = EP - E
    srcp = jnp.concatenate([src, jnp.zeros((pad,), jnp.int32)]
                           ).reshape(NSUB, NCHUNK, CH)
    dstp = jnp.concatenate([dst, jnp.full((pad,), N, jnp.int32)]
                           ).reshape(NSUB, NCHUNK, CH)

    parts1 = _sc_conv1(h0, srcp, dstp, jnp.zeros((NP, 16), f32))

    h1p, cntcol = pl.pallas_call(
        _tc_mid_body,
        out_shape=(jax.ShapeDtypeStruct((NCORE, N, 64), f32),
                   jax.ShapeDtypeStruct((N, 1), f32)),
    )(parts1, h0, Wl1.T, Wr1.T, b1)

    parts2 = _sc_conv2(h1p, srcp, dstp, jnp.zeros((NP, 64), f32))

    sl, sr = pl.pallas_call(
        _tc_out_body,
        out_shape=(jax.ShapeDtypeStruct((N, 1), f32),
                   jax.ShapeDtypeStruct((N, 1), f32)),
    )(parts2, cntcol, h1p, Wl2.T, Wr2.T, b2,
      Wp[0, :128].reshape(128, 1), Wp[0, 128:].reshape(128, 1), bp)

    raw2, sig2 = _sc_head(sl.reshape(N), sr.reshape(N),
                          src.reshape(NW, EPW), dst.reshape(NW, EPW))
    return raw2.reshape(E), sig2.reshape(E)


# speedup vs baseline: 1.4429x; 1.4429x over previous
import functools

import jax
import jax.numpy as jnp
from jax import lax
from jax.experimental import pallas as pl
from jax.experimental.pallas import tpu as pltpu
from jax.experimental.pallas import tpu_sc as plsc

N = 10000
E = 320000
NCORE = 2
NSUB = 16
NW = NCORE * NSUB
CH = 128
NCHUNK = 160
EP = NSUB * NCHUNK * CH
NP = 10112
EPW = E // NW
NSLOT = 5
CS2 = 2
ZROWS = NP // NSUB
OROWS = 624
OTAIL = N - OROWS * NSUB

_mesh = plsc.VectorSubcoreMesh(core_axis_name="c", subcore_axis_name="s")
_params = pltpu.CompilerParams(use_tc_tiling_on_sc=False)
_params_head = pltpu.CompilerParams(use_tc_tiling_on_sc=False,
                                    needs_layout_passes=False)


def _stage_and_zero(srcp_hbm, dstp_hbm, z_hbm, srcv, dstv, aggsh, sid):
    pltpu.sync_copy(srcp_hbm.at[sid], srcv)
    pltpu.sync_copy(dstp_hbm.at[sid], dstv)
    pltpu.sync_copy(z_hbm.at[pl.ds(sid * ZROWS, ZROWS)],
                    aggsh.at[pl.ds(sid * ZROWS, ZROWS)])


def _copy_out(aggsh, out_hbm, cid, sid):
    pltpu.sync_copy(aggsh.at[pl.ds(sid * OROWS, OROWS)],
                    out_hbm.at[cid, pl.ds(sid * OROWS, OROWS)])

    @pl.when(sid == 0)
    def _():
        pltpu.sync_copy(aggsh.at[pl.ds(OROWS * NSUB, OTAIL)],
                        out_hbm.at[cid, pl.ds(OROWS * NSUB, OTAIL)])


def _gather_scatter_loop(src_hbm, srcv, dstv, bufs, gsems, ssems, aggsh):
    del ssems
    for k in range(NSLOT):
        pltpu.async_copy(src_hbm.at[srcv.at[k]], bufs[k], gsems[k])

    def body(t, carry):
        base = t * NSLOT
        for k in range(NSLOT):
            pltpu.make_async_copy(src_hbm.at[srcv.at[base + k]],
                                  bufs[k], gsems[k]).wait()
            pltpu.sync_copy(bufs[k], aggsh.at[dstv.at[base + k]], add=True)
            nxt = base + k + NSLOT

            @pl.when(nxt < NCHUNK)
            def _():
                pltpu.async_copy(src_hbm.at[srcv.at[nxt]], bufs[k],
                                 gsems[k])
        return carry

    lax.fori_loop(0, NCHUNK // NSLOT, body, 0)


@functools.partial(
    pl.kernel,
    out_type=jax.ShapeDtypeStruct((NCORE, N, 16), jnp.float32),
    mesh=_mesh,
    scratch_types=[
        pltpu.VMEM((NCHUNK, CH), jnp.int32),
        pltpu.VMEM((NCHUNK, CH), jnp.int32),
    ]
    + [pltpu.VMEM((CH, 16), jnp.float32) for _ in range(NSLOT)]
    + [pltpu.VMEM_SHARED((NP, 16), jnp.float32)]
    + [pltpu.SemaphoreType.DMA for _ in range(2 * NSLOT)],
    compiler_params=_params,
)
def _sc_conv1(h0_hbm, srcp_hbm, dstp_hbm, z_hbm, out_hbm,
              srcv, dstv, *rest):
    bufs = rest[:NSLOT]
    aggsh = rest[NSLOT]
    gsems = rest[NSLOT + 1:2 * NSLOT + 1]
    ssems = rest[2 * NSLOT + 1:]
    cid = lax.axis_index("c")
    sid = lax.axis_index("s")
    _stage_and_zero(srcp_hbm, dstp_hbm, z_hbm, srcv, dstv, aggsh, sid)
    plsc.subcore_barrier()

    @pl.when(cid == 0)
    def _():
        _gather_scatter_loop(h0_hbm, srcv, dstv, bufs, gsems, ssems, aggsh)

    @pl.when(cid == 1)
    def _():
        ones_col = jnp.where(lax.iota(jnp.int32, 16) == 0, 1.0, 0.0)

        def fill(r, carry):
            bufs[0][r] = ones_col
            return carry

        lax.fori_loop(0, CH, fill, 0)

        for k in range(NSLOT):
            pltpu.async_copy(bufs[0], aggsh.at[dstv.at[k]], ssems[k],
                             add=True)

        def body(t, carry):
            base = t * NSLOT
            for k in range(NSLOT):
                j = base + k
                pltpu.make_async_copy(bufs[0], aggsh.at[dstv.at[0]],
                                      ssems[k]).wait()
                nxt = j + NSLOT

                @pl.when(nxt < NCHUNK)
                def _():
                    pltpu.async_copy(bufs[0], aggsh.at[dstv.at[nxt]],
                                     ssems[k], add=True)
            return carry

        lax.fori_loop(0, NCHUNK // NSLOT, body, 0)

    plsc.subcore_barrier()
    _copy_out(aggsh, out_hbm, cid, sid)


@functools.partial(
    pl.kernel,
    out_type=jax.ShapeDtypeStruct((NCORE, N, 64), jnp.float32),
    mesh=_mesh,
    scratch_types=[
        pltpu.VMEM((NCHUNK, CH), jnp.int32),
    ]
    + [pltpu.VMEM((CS2, CH), jnp.int32)]
    + [pltpu.VMEM((CH, 64), jnp.float32) for _ in range(CS2)]
    + [pltpu.VMEM_SHARED((N, 64), jnp.float32)]
    + [pltpu.VMEM_SHARED((NP, 64), jnp.float32)]
    + [pltpu.SemaphoreType.DMA for _ in range(2 * CS2)],
    compiler_params=_params,
)
def _sc_conv2(h1p_hbm, srcp_hbm, dstp_hbm, z_hbm, out_hbm,
              srcv, *rest):
    didx = rest[0]
    bufs = rest[1:CS2 + 1]
    h1sh = rest[CS2 + 1]
    aggsh = rest[CS2 + 2]
    gsems = rest[CS2 + 3:2 * CS2 + 3]
    dsems = rest[2 * CS2 + 3:]
    cid = lax.axis_index("c")
    sid = lax.axis_index("s")
    pltpu.sync_copy(srcp_hbm.at[sid], srcv)
    pltpu.sync_copy(h1p_hbm.at[cid, pl.ds(sid * OROWS, OROWS)],
                    h1sh.at[pl.ds(sid * OROWS, OROWS)])
    pltpu.sync_copy(z_hbm.at[pl.ds(sid * ZROWS, ZROWS)],
                    aggsh.at[pl.ds(sid * ZROWS, ZROWS)])

    @pl.when(sid == 0)
    def _():
        pltpu.sync_copy(h1p_hbm.at[cid, pl.ds(OROWS * NSUB, OTAIL)],
                        h1sh.at[pl.ds(OROWS * NSUB, OTAIL)])

    plsc.subcore_barrier()
    for k in range(CS2):
        pltpu.async_copy(dstp_hbm.at[sid, k], didx.at[k], dsems[k])
        pltpu.async_copy(h1sh.at[srcv.at[k]], bufs[k], gsems[k])

    def body(t, carry):
        base = t * CS2
        for k in range(CS2):
            j = base + k
            pltpu.make_async_copy(h1sh.at[srcv.at[j]],
                                  bufs[k], gsems[k]).wait()
            pltpu.make_async_copy(dstp_hbm.at[sid, j],
                                  didx.at[k], dsems[k]).wait()
            pltpu.sync_copy(bufs[k], aggsh.at[didx.at[k]], add=True)
            nxt = j + CS2

            @pl.when(nxt < NCHUNK)
            def _():
                pltpu.async_copy(dstp_hbm.at[sid, nxt], didx.at[k],
                                 dsems[k])
                pltpu.async_copy(h1sh.at[srcv.at[nxt]], bufs[k], gsems[k])
        return carry

    lax.fori_loop(0, NCHUNK // CS2, body, 0)
    plsc.subcore_barrier()
    _copy_out(aggsh, out_hbm, cid, sid)


@functools.partial(
    pl.kernel,
    out_type=(jax.ShapeDtypeStruct((NW, EPW), jnp.float32),
              jax.ShapeDtypeStruct((NW, EPW), jnp.float32)),
    mesh=_mesh,
    scratch_types=[
        pltpu.VMEM((EPW,), jnp.int32),
        pltpu.VMEM((EPW,), jnp.int32),
        pltpu.VMEM((N,), jnp.float32),
        pltpu.VMEM((N,), jnp.float32),
        pltpu.VMEM((EPW,), jnp.float32),
        pltpu.VMEM((EPW,), jnp.float32),
    ],
    compiler_params=_params_head,
)
def _sc_head(sl_hbm, sr_hbm, src_hbm, dst_hbm, raw_hbm, sig_hbm,
             srcv, dstv, slv, srv, rawv, sigv):
    cid = lax.axis_index("c")
    sid = lax.axis_index("s")
    wid = sid * NCORE + cid
    pltpu.sync_copy(src_hbm.at[wid], srcv)
    pltpu.sync_copy(dst_hbm.at[wid], dstv)
    pltpu.sync_copy(sl_hbm, slv)
    pltpu.sync_copy(sr_hbm, srv)

    def body(i, carry):
        o = i * 16
        s = srcv[pl.ds(o, 16)]
        d = dstv[pl.ds(o, 16)]
        raw = plsc.load_gather(slv, [s]) + plsc.load_gather(srv, [d])
        rawv[pl.ds(o, 16)] = raw
        sigv[pl.ds(o, 16)] = 1.0 / (1.0 + jnp.exp(-raw))
        return carry

    lax.fori_loop(0, EPW // 16, body, 0)
    pltpu.sync_copy(rawv, raw_hbm.at[wid])
    pltpu.sync_copy(sigv, sig_hbm.at[wid])


def _tc_lin0_body(x_ref, w_ref, b_ref, o_ref):
    o_ref[...] = (jnp.dot(x_ref[...], w_ref[...],
                          preferred_element_type=jnp.float32)
                  + b_ref[...][None, :])


def _tc_mid_body(p_ref, h0_ref, wl_ref, wr_ref, b_ref, h1p_ref, cnt_ref):
    cnt = jnp.maximum(p_ref[1][:, :1], 1.0)
    mean = p_ref[0] / cnt
    h1 = (jnp.dot(mean, wl_ref[...], preferred_element_type=jnp.float32)
          + jnp.dot(h0_ref[...], wr_ref[...],
                    preferred_element_type=jnp.float32)
          + b_ref[...][None, :])
    h1 = jnp.maximum(h1, 0.0)
    h1p_ref[0] = h1[:, :64]
    h1p_ref[1] = h1[:, 64:]
    cnt_ref[...] = cnt


def _tc_out_body(p_ref, cnt_ref, h1p_ref, wl_ref, wr_ref, b_ref,
                 wpl_ref, wpr_ref, bp_ref, sl_ref, sr_ref):
    cnt = cnt_ref[...]
    h2 = (jnp.dot(p_ref[0] / cnt, wl_ref[...][:64],
                  preferred_element_type=jnp.float32)
          + jnp.dot(p_ref[1] / cnt, wl_ref[...][64:],
                    preferred_element_type=jnp.float32)
          + jnp.dot(h1p_ref[0], wr_ref[...][:64],
                    preferred_element_type=jnp.float32)
          + jnp.dot(h1p_ref[1], wr_ref[...][64:],
                    preferred_element_type=jnp.float32)
          + b_ref[...][None, :])
    sl_ref[...] = (jnp.dot(h2, wpl_ref[...],
                           preferred_element_type=jnp.float32)
                   + bp_ref[...][None, :])
    sr_ref[...] = jnp.dot(h2, wpr_ref[...],
                          preferred_element_type=jnp.float32)


def kernel(x, edge_index, edge_attr, W0, b0, Wl1, Wr1, b1,
           Wl2, Wr2, b2, Wp, bp):
    f32 = jnp.float32
    src = edge_index[0]
    dst = edge_index[1]

    h0 = pl.pallas_call(
        _tc_lin0_body,
        out_shape=jax.ShapeDtypeStruct((N, 16), f32),
    )(x, W0.T, b0)

    pad = EP - E
    srcp = jnp.concatenate([src, jnp.zeros((pad,), jnp.int32)]
                           ).reshape(NSUB, NCHUNK, CH)
    dstp = jnp.concatenate([dst, jnp.full((pad,), N, jnp.int32)]
                           ).reshape(NSUB, NCHUNK, CH)

    parts1 = _sc_conv1(h0, srcp, dstp, jnp.zeros((NP, 16), f32))

    h1p, cntcol = pl.pallas_call(
        _tc_mid_body,
        out_shape=(jax.ShapeDtypeStruct((NCORE, N, 64), f32),
                   jax.ShapeDtypeStruct((N, 1), f32)),
    )(parts1, h0, Wl1.T, Wr1.T, b1)

    parts2 = _sc_conv2(h1p, srcp, dstp, jnp.zeros((NP, 64), f32))

    sl, sr = pl.pallas_call(
        _tc_out_body,
        out_shape=(jax.ShapeDtypeStruct((N, 1), f32),
                   jax.ShapeDtypeStruct((N, 1), f32)),
    )(parts2, cntcol, h1p, Wl2.T, Wr2.T, b2,
      Wp[0, :128].reshape(128, 1), Wp[0, 128:].reshape(128, 1), bp)

    raw2, sig2 = _sc_head(sl.reshape(N), sr.reshape(N),
                          src.reshape(NW, EPW), dst.reshape(NW, EPW))
    return raw2.reshape(E), sig2.reshape(E)


# speedup vs baseline: 1.5867x; 1.0997x over previous
import functools

import jax
import jax.numpy as jnp
from jax import lax
from jax.experimental import pallas as pl
from jax.experimental.pallas import tpu as pltpu
from jax.experimental.pallas import tpu_sc as plsc

N = 10000
E = 320000
NCORE = 2
NSUB = 16
NW = NCORE * NSUB
CH = 128
NCHUNK = 160
EP = NSUB * NCHUNK * CH
NP = 10112
EPW = E // NW
NSLOT = 4
NCHUNK1 = NCHUNK // 2
CS2 = 2
ZROWS = NP // NSUB
OROWS = 624
OTAIL = N - OROWS * NSUB

_mesh = plsc.VectorSubcoreMesh(core_axis_name="c", subcore_axis_name="s")
_params = pltpu.CompilerParams(use_tc_tiling_on_sc=False)
_params_head = pltpu.CompilerParams(use_tc_tiling_on_sc=False,
                                    needs_layout_passes=False)


def _copy_out(aggsh, out_hbm, cid, sid):
    pltpu.sync_copy(aggsh.at[pl.ds(sid * OROWS, OROWS)],
                    out_hbm.at[cid, pl.ds(sid * OROWS, OROWS)])

    @pl.when(sid == 0)
    def _():
        pltpu.sync_copy(aggsh.at[pl.ds(OROWS * NSUB, OTAIL)],
                        out_hbm.at[cid, pl.ds(OROWS * NSUB, OTAIL)])


@functools.partial(
    pl.kernel,
    out_type=(jax.ShapeDtypeStruct((NCORE, N, 16), jnp.float32),
              jax.ShapeDtypeStruct((NCORE, N, 16), jnp.float32)),
    mesh=_mesh,
    scratch_types=[
        pltpu.VMEM((NCHUNK1, CH), jnp.int32),
        pltpu.VMEM((NCHUNK1, CH), jnp.int32),
    ]
    + [pltpu.VMEM((CH, 16), jnp.float32) for _ in range(NSLOT)]
    + [pltpu.VMEM((CH, 16), jnp.float32)]
    + [pltpu.VMEM_SHARED((N, 16), jnp.float32)]
    + [pltpu.VMEM_SHARED((NP, 16), jnp.float32)]
    + [pltpu.VMEM_SHARED((NP, 16), jnp.float32)]
    + [pltpu.SemaphoreType.DMA for _ in range(NSLOT)],
    compiler_params=_params,
)
def _sc_conv1(h0_hbm, srcp_hbm, dstp_hbm, z_hbm, fout_hbm, cout_hbm,
              srcv, dstv, *rest):
    bufs = rest[:NSLOT]
    cbuf = rest[NSLOT]
    h0sh = rest[NSLOT + 1]
    featsh = rest[NSLOT + 2]
    cntsh = rest[NSLOT + 3]
    gsems = rest[NSLOT + 4:]
    cid = lax.axis_index("c")
    sid = lax.axis_index("s")
    c0 = cid * NCHUNK1
    pltpu.sync_copy(srcp_hbm.at[sid, pl.ds(c0, NCHUNK1)], srcv)
    pltpu.sync_copy(dstp_hbm.at[sid, pl.ds(c0, NCHUNK1)], dstv)
    pltpu.sync_copy(h0_hbm.at[pl.ds(sid * OROWS, OROWS)],
                    h0sh.at[pl.ds(sid * OROWS, OROWS)])
    pltpu.sync_copy(z_hbm.at[pl.ds(sid * ZROWS, ZROWS)],
                    featsh.at[pl.ds(sid * ZROWS, ZROWS)])
    pltpu.sync_copy(z_hbm.at[pl.ds(sid * ZROWS, ZROWS)],
                    cntsh.at[pl.ds(sid * ZROWS, ZROWS)])

    @pl.when(sid == 0)
    def _():
        pltpu.sync_copy(h0_hbm.at[pl.ds(OROWS * NSUB, OTAIL)],
                        h0sh.at[pl.ds(OROWS * NSUB, OTAIL)])

    ones_col = jnp.where(lax.iota(jnp.int32, 16) == 0, 1.0, 0.0)

    def fill(r, carry):
        cbuf[r] = ones_col
        return carry

    lax.fori_loop(0, CH, fill, 0)
    plsc.subcore_barrier()
    for k in range(NSLOT):
        pltpu.async_copy(h0sh.at[srcv.at[k]], bufs[k], gsems[k])

    def body(t, carry):
        base = t * NSLOT
        for k in range(NSLOT):
            j = base + k
            pltpu.make_async_copy(h0sh.at[srcv.at[j]],
                                  bufs[k], gsems[k]).wait()
            pltpu.sync_copy(bufs[k], featsh.at[dstv.at[j]], add=True)
            pltpu.sync_copy(cbuf, cntsh.at[dstv.at[j]], add=True)
            nxt = j + NSLOT

            @pl.when(nxt < NCHUNK1)
            def _():
                pltpu.async_copy(h0sh.at[srcv.at[nxt]], bufs[k], gsems[k])
        return carry

    lax.fori_loop(0, NCHUNK1 // NSLOT, body, 0)
    plsc.subcore_barrier()
    _copy_out(featsh, fout_hbm, cid, sid)
    _copy_out(cntsh, cout_hbm, cid, sid)


@functools.partial(
    pl.kernel,
    out_type=jax.ShapeDtypeStruct((NCORE, N, 64), jnp.float32),
    mesh=_mesh,
    scratch_types=[
        pltpu.VMEM((NCHUNK, CH), jnp.int32),
    ]
    + [pltpu.VMEM((CS2, CH), jnp.int32)]
    + [pltpu.VMEM((CH, 64), jnp.float32) for _ in range(CS2)]
    + [pltpu.VMEM_SHARED((N, 64), jnp.float32)]
    + [pltpu.VMEM_SHARED((NP, 64), jnp.float32)]
    + [pltpu.SemaphoreType.DMA for _ in range(2 * CS2)],
    compiler_params=_params,
)
def _sc_conv2(h1p_hbm, srcp_hbm, dstp_hbm, z_hbm, out_hbm,
              srcv, *rest):
    didx = rest[0]
    bufs = rest[1:CS2 + 1]
    h1sh = rest[CS2 + 1]
    aggsh = rest[CS2 + 2]
    gsems = rest[CS2 + 3:2 * CS2 + 3]
    dsems = rest[2 * CS2 + 3:]
    cid = lax.axis_index("c")
    sid = lax.axis_index("s")
    pltpu.sync_copy(srcp_hbm.at[sid], srcv)
    pltpu.sync_copy(h1p_hbm.at[cid, pl.ds(sid * OROWS, OROWS)],
                    h1sh.at[pl.ds(sid * OROWS, OROWS)])
    pltpu.sync_copy(z_hbm.at[pl.ds(sid * ZROWS, ZROWS)],
                    aggsh.at[pl.ds(sid * ZROWS, ZROWS)])

    @pl.when(sid == 0)
    def _():
        pltpu.sync_copy(h1p_hbm.at[cid, pl.ds(OROWS * NSUB, OTAIL)],
                        h1sh.at[pl.ds(OROWS * NSUB, OTAIL)])

    plsc.subcore_barrier()
    for k in range(CS2):
        pltpu.async_copy(dstp_hbm.at[sid, k], didx.at[k], dsems[k])
        pltpu.async_copy(h1sh.at[srcv.at[k]], bufs[k], gsems[k])

    def body(t, carry):
        base = t * CS2
        for k in range(CS2):
            j = base + k
            pltpu.make_async_copy(h1sh.at[srcv.at[j]],
                                  bufs[k], gsems[k]).wait()
            pltpu.make_async_copy(dstp_hbm.at[sid, j],
                                  didx.at[k], dsems[k]).wait()
            pltpu.sync_copy(bufs[k], aggsh.at[didx.at[k]], add=True)
            nxt = j + CS2

            @pl.when(nxt < NCHUNK)
            def _():
                pltpu.async_copy(dstp_hbm.at[sid, nxt], didx.at[k],
                                 dsems[k])
                pltpu.async_copy(h1sh.at[srcv.at[nxt]], bufs[k], gsems[k])
        return carry

    lax.fori_loop(0, NCHUNK // CS2, body, 0)
    plsc.subcore_barrier()
    _copy_out(aggsh, out_hbm, cid, sid)


@functools.partial(
    pl.kernel,
    out_type=(jax.ShapeDtypeStruct((NW, EPW), jnp.float32),
              jax.ShapeDtypeStruct((NW, EPW), jnp.float32)),
    mesh=_mesh,
    scratch_types=[
        pltpu.VMEM((EPW,), jnp.int32),
        pltpu.VMEM((EPW,), jnp.int32),
        pltpu.VMEM((N,), jnp.float32),
        pltpu.VMEM((N,), jnp.float32),
        pltpu.VMEM((EPW,), jnp.float32),
        pltpu.VMEM((EPW,), jnp.float32),
    ],
    compiler_params=_params_head,
)
def _sc_head(sl_hbm, sr_hbm, src_hbm, dst_hbm, raw_hbm, sig_hbm,
             srcv, dstv, slv, srv, rawv, sigv):
    cid = lax.axis_index("c")
    sid = lax.axis_index("s")
    wid = sid * NCORE + cid
    pltpu.sync_copy(src_hbm.at[wid], srcv)
    pltpu.sync_copy(dst_hbm.at[wid], dstv)
    pltpu.sync_copy(sl_hbm, slv)
    pltpu.sync_copy(sr_hbm, srv)

    def body(i, carry):
        o = i * 16
        s = srcv[pl.ds(o, 16)]
        d = dstv[pl.ds(o, 16)]
        raw = plsc.load_gather(slv, [s]) + plsc.load_gather(srv, [d])
        rawv[pl.ds(o, 16)] = raw
        sigv[pl.ds(o, 16)] = 1.0 / (1.0 + jnp.exp(-raw))
        return carry

    lax.fori_loop(0, EPW // 16, body, 0)
    pltpu.sync_copy(rawv, raw_hbm.at[wid])
    pltpu.sync_copy(sigv, sig_hbm.at[wid])


def _tc_lin0_body(x_ref, w_ref, b_ref, o_ref):
    o_ref[...] = (jnp.dot(x_ref[...], w_ref[...],
                          preferred_element_type=jnp.float32)
                  + b_ref[...][None, :])


def _tc_mid_body(f_ref, c_ref, h0_ref, wl_ref, wr_ref, b_ref,
                 h1p_ref, cnt_ref):
    cnt = jnp.maximum(c_ref[0][:, :1] + c_ref[1][:, :1], 1.0)
    mean = (f_ref[0] + f_ref[1]) / cnt
    h1 = (jnp.dot(mean, wl_ref[...], preferred_element_type=jnp.float32)
          + jnp.dot(h0_ref[...], wr_ref[...],
                    preferred_element_type=jnp.float32)
          + b_ref[...][None, :])
    h1 = jnp.maximum(h1, 0.0)
    h1p_ref[0] = h1[:, :64]
    h1p_ref[1] = h1[:, 64:]
    cnt_ref[...] = cnt


def _tc_out_body(p_ref, cnt_ref, h1p_ref, wl_ref, wr_ref, b_ref,
                 wpl_ref, wpr_ref, bp_ref, sl_ref, sr_ref):
    cnt = cnt_ref[...]
    h2 = (jnp.dot(p_ref[0] / cnt, wl_ref[...][:64],
                  preferred_element_type=jnp.float32)
          + jnp.dot(p_ref[1] / cnt, wl_ref[...][64:],
                    preferred_element_type=jnp.float32)
          + jnp.dot(h1p_ref[0], wr_ref[...][:64],
                    preferred_element_type=jnp.float32)
          + jnp.dot(h1p_ref[1], wr_ref[...][64:],
                    preferred_element_type=jnp.float32)
          + b_ref[...][None, :])
    sl_ref[...] = (jnp.dot(h2, wpl_ref[...],
                           preferred_element_type=jnp.float32)
                   + bp_ref[...][None, :])
    sr_ref[...] = jnp.dot(h2, wpr_ref[...],
                          preferred_element_type=jnp.float32)


def kernel(x, edge_index, edge_attr, W0, b0, Wl1, Wr1, b1,
           Wl2, Wr2, b2, Wp, bp):
    f32 = jnp.float32
    src = edge_index[0]
    dst = edge_index[1]

    h0 = pl.pallas_call(
        _tc_lin0_body,
        out_shape=jax.ShapeDtypeStruct((N, 16), f32),
    )(x, W0.T, b0)

    pad = EP - E
    srcp = jnp.concatenate([src, jnp.zeros((pad,), jnp.int32)]
                           ).reshape(NSUB, NCHUNK, CH)
    dstp = jnp.concatenate([dst, jnp.full((pad,), N, jnp.int32)]
                           ).reshape(NSUB, NCHUNK, CH)

    fparts, cparts = _sc_conv1(h0, srcp, dstp, jnp.zeros((NP, 16), f32))

    h1p, cntcol = pl.pallas_call(
        _tc_mid_body,
        out_shape=(jax.ShapeDtypeStruct((NCORE, N, 64), f32),
                   jax.ShapeDtypeStruct((N, 1), f32)),
    )(fparts, cparts, h0, Wl1.T, Wr1.T, b1)

    parts2 = _sc_conv2(h1p, srcp, dstp, jnp.zeros((NP, 64), f32))

    sl, sr = pl.pallas_call(
        _tc_out_body,
        out_shape=(jax.ShapeDtypeStruct((N, 1), f32),
                   jax.ShapeDtypeStruct((N, 1), f32)),
    )(parts2, cntcol, h1p, Wl2.T, Wr2.T, b2,
      Wp[0, :128].reshape(128, 1), Wp[0, 128:].reshape(128, 1), bp)

    raw2, sig2 = _sc_head(sl.reshape(N), sr.reshape(N),
                          src.reshape(NW, EPW), dst.reshape(NW, EPW))
    return raw2.reshape(E), sig2.reshape(E)


# speedup vs baseline: 1.6883x; 1.0640x over previous
import functools

import jax
import jax.numpy as jnp
from jax import lax
from jax.experimental import pallas as pl
from jax.experimental.pallas import tpu as pltpu
from jax.experimental.pallas import tpu_sc as plsc

N = 10000
E = 320000
NCORE = 2
NSUB = 16
NW = NCORE * NSUB
CH = 128
ECH = E // CH
NP = 10112
EPW = E // NW
C1B = 78
C1XT = 14
C2B = 156
C2XT = 12
NSLOT = 3
CS2 = 2
ZROWS = NP // NSUB
OROWS = 624
OTAIL = N - OROWS * NSUB

_mesh = plsc.VectorSubcoreMesh(core_axis_name="c", subcore_axis_name="s")
_params = pltpu.CompilerParams(use_tc_tiling_on_sc=False)
_params_head = pltpu.CompilerParams(use_tc_tiling_on_sc=False,
                                    needs_layout_passes=False)


def _copy_out(aggsh, out_hbm, cid, sid):
    pltpu.sync_copy(aggsh.at[pl.ds(sid * OROWS, OROWS)],
                    out_hbm.at[cid, pl.ds(sid * OROWS, OROWS)])

    @pl.when(sid == 0)
    def _():
        pltpu.sync_copy(aggsh.at[pl.ds(OROWS * NSUB, OTAIL)],
                        out_hbm.at[cid, pl.ds(OROWS * NSUB, OTAIL)])


@functools.partial(
    pl.kernel,
    out_type=(jax.ShapeDtypeStruct((NCORE, N, 16), jnp.float32),
              jax.ShapeDtypeStruct((NCORE, N, 16), jnp.float32)),
    mesh=_mesh,
    scratch_types=[
        pltpu.VMEM((C1B + 1, CH), jnp.int32),
        pltpu.VMEM((C1B + 1, CH), jnp.int32),
    ]
    + [pltpu.VMEM((CH, 16), jnp.float32) for _ in range(NSLOT)]
    + [pltpu.VMEM((CH, 16), jnp.float32)]
    + [pltpu.VMEM_SHARED((N, 16), jnp.float32)]
    + [pltpu.VMEM_SHARED((NP, 16), jnp.float32)]
    + [pltpu.VMEM_SHARED((NP, 16), jnp.float32)]
    + [pltpu.SemaphoreType.DMA for _ in range(NSLOT)],
    compiler_params=_params,
)
def _sc_conv1(h0_hbm, ei_hbm, z_hbm, fout_hbm, cout_hbm,
              srcv, dstv, *rest):
    bufs = rest[:NSLOT]
    cbuf = rest[NSLOT]
    h0sh = rest[NSLOT + 1]
    featsh = rest[NSLOT + 2]
    cntsh = rest[NSLOT + 3]
    gsems = rest[NSLOT + 4:]
    cid = lax.axis_index("c")
    sid = lax.axis_index("s")
    base = (cid * (ECH // 2) + sid * C1B
            + jnp.maximum(sid - C1XT, 0).astype(jnp.int32))
    pltpu.sync_copy(ei_hbm.at[0, pl.ds(base, C1B)],
                    srcv.at[pl.ds(0, C1B)])
    pltpu.sync_copy(ei_hbm.at[1, pl.ds(base, C1B)],
                    dstv.at[pl.ds(0, C1B)])

    @pl.when(sid >= C1XT)
    def _():
        pltpu.sync_copy(ei_hbm.at[0, pl.ds(base + C1B, 1)],
                        srcv.at[pl.ds(C1B, 1)])
        pltpu.sync_copy(ei_hbm.at[1, pl.ds(base + C1B, 1)],
                        dstv.at[pl.ds(C1B, 1)])

    pltpu.sync_copy(h0_hbm.at[pl.ds(sid * OROWS, OROWS)],
                    h0sh.at[pl.ds(sid * OROWS, OROWS)])
    pltpu.sync_copy(z_hbm.at[pl.ds(sid * ZROWS, ZROWS)],
                    featsh.at[pl.ds(sid * ZROWS, ZROWS)])
    pltpu.sync_copy(z_hbm.at[pl.ds(sid * ZROWS, ZROWS)],
                    cntsh.at[pl.ds(sid * ZROWS, ZROWS)])

    @pl.when(sid == 0)
    def _():
        pltpu.sync_copy(h0_hbm.at[pl.ds(OROWS * NSUB, OTAIL)],
                        h0sh.at[pl.ds(OROWS * NSUB, OTAIL)])

    ones_col = jnp.where(lax.iota(jnp.int32, 16) == 0, 1.0, 0.0)

    def fill(r, carry):
        cbuf[r] = ones_col
        return carry

    lax.fori_loop(0, CH, fill, 0)
    plsc.subcore_barrier()
    for k in range(NSLOT):
        pltpu.async_copy(h0sh.at[srcv.at[k]], bufs[k], gsems[k])

    def body(t, carry):
        b0 = t * NSLOT
        for k in range(NSLOT):
            j = b0 + k
            pltpu.make_async_copy(h0sh.at[srcv.at[j]],
                                  bufs[k], gsems[k]).wait()
            pltpu.sync_copy(bufs[k], featsh.at[dstv.at[j]], add=True)
            pltpu.sync_copy(cbuf, cntsh.at[dstv.at[j]], add=True)
            nxt = j + NSLOT

            @pl.when(nxt < C1B)
            def _():
                pltpu.async_copy(h0sh.at[srcv.at[nxt]], bufs[k], gsems[k])
        return carry

    lax.fori_loop(0, C1B // NSLOT, body, 0)

    @pl.when(sid >= C1XT)
    def _():
        pltpu.async_copy(h0sh.at[srcv.at[C1B]], bufs[0], gsems[0]).wait()
        pltpu.sync_copy(bufs[0], featsh.at[dstv.at[C1B]], add=True)
        pltpu.sync_copy(cbuf, cntsh.at[dstv.at[C1B]], add=True)

    plsc.subcore_barrier()
    _copy_out(featsh, fout_hbm, cid, sid)
    _copy_out(cntsh, cout_hbm, cid, sid)


@functools.partial(
    pl.kernel,
    out_type=jax.ShapeDtypeStruct((NCORE, N, 64), jnp.float32),
    mesh=_mesh,
    scratch_types=[
        pltpu.VMEM((C2B + 1, CH), jnp.int32),
    ]
    + [pltpu.VMEM((CS2, CH), jnp.int32)]
    + [pltpu.VMEM((CH, 64), jnp.float32) for _ in range(CS2)]
    + [pltpu.VMEM_SHARED((N, 64), jnp.float32)]
    + [pltpu.VMEM_SHARED((NP, 64), jnp.float32)]
    + [pltpu.SemaphoreType.DMA for _ in range(2 * CS2)],
    compiler_params=_params,
)
def _sc_conv2(h1p_hbm, ei_hbm, z_hbm, out_hbm,
              srcv, *rest):
    didx = rest[0]
    bufs = rest[1:CS2 + 1]
    h1sh = rest[CS2 + 1]
    aggsh = rest[CS2 + 2]
    gsems = rest[CS2 + 3:2 * CS2 + 3]
    dsems = rest[2 * CS2 + 3:]
    cid = lax.axis_index("c")
    sid = lax.axis_index("s")
    base = sid * C2B + jnp.maximum(sid - C2XT, 0).astype(jnp.int32)
    pltpu.sync_copy(ei_hbm.at[0, pl.ds(base, C2B)],
                    srcv.at[pl.ds(0, C2B)])

    @pl.when(sid >= C2XT)
    def _():
        pltpu.sync_copy(ei_hbm.at[0, pl.ds(base + C2B, 1)],
                        srcv.at[pl.ds(C2B, 1)])

    pltpu.sync_copy(h1p_hbm.at[cid, pl.ds(sid * OROWS, OROWS)],
                    h1sh.at[pl.ds(sid * OROWS, OROWS)])
    pltpu.sync_copy(z_hbm.at[pl.ds(sid * ZROWS, ZROWS)],
                    aggsh.at[pl.ds(sid * ZROWS, ZROWS)])

    @pl.when(sid == 0)
    def _():
        pltpu.sync_copy(h1p_hbm.at[cid, pl.ds(OROWS * NSUB, OTAIL)],
                        h1sh.at[pl.ds(OROWS * NSUB, OTAIL)])

    plsc.subcore_barrier()
    for k in range(CS2):
        pltpu.async_copy(ei_hbm.at[1, base + k], didx.at[k], dsems[k])
        pltpu.async_copy(h1sh.at[srcv.at[k]], bufs[k], gsems[k])

    def body(t, carry):
        b0 = t * CS2
        for k in range(CS2):
            j = b0 + k
            pltpu.make_async_copy(h1sh.at[srcv.at[j]],
                                  bufs[k], gsems[k]).wait()
            pltpu.make_async_copy(ei_hbm.at[1, base + j],
                                  didx.at[k], dsems[k]).wait()
            pltpu.sync_copy(bufs[k], aggsh.at[didx.at[k]], add=True)
            nxt = j + CS2

            @pl.when(nxt < C2B)
            def _():
                pltpu.async_copy(ei_hbm.at[1, base + nxt], didx.at[k],
                                 dsems[k])
                pltpu.async_copy(h1sh.at[srcv.at[nxt]], bufs[k], gsems[k])
        return carry

    lax.fori_loop(0, C2B // CS2, body, 0)

    @pl.when(sid >= C2XT)
    def _():
        pltpu.async_copy(ei_hbm.at[1, base + C2B], didx.at[0],
                         dsems[0]).wait()
        pltpu.async_copy(h1sh.at[srcv.at[C2B]], bufs[0], gsems[0]).wait()
        pltpu.sync_copy(bufs[0], aggsh.at[didx.at[0]], add=True)

    plsc.subcore_barrier()
    _copy_out(aggsh, out_hbm, cid, sid)


@functools.partial(
    pl.kernel,
    out_type=(jax.ShapeDtypeStruct((E,), jnp.float32),
              jax.ShapeDtypeStruct((E,), jnp.float32)),
    mesh=_mesh,
    scratch_types=[
        pltpu.VMEM((EPW,), jnp.int32),
        pltpu.VMEM((EPW,), jnp.int32),
        pltpu.VMEM((N,), jnp.float32),
        pltpu.VMEM((N,), jnp.float32),
        pltpu.VMEM((EPW,), jnp.float32),
        pltpu.VMEM((EPW,), jnp.float32),
    ],
    compiler_params=_params_head,
)
def _sc_head(sl_hbm, sr_hbm, ei_hbm, raw_hbm, sig_hbm,
             srcv, dstv, slv, srv, rawv, sigv):
    cid = lax.axis_index("c")
    sid = lax.axis_index("s")
    wid = sid * NCORE + cid
    pltpu.sync_copy(ei_hbm.at[0, pl.ds(wid * EPW, EPW)], srcv)
    pltpu.sync_copy(ei_hbm.at[1, pl.ds(wid * EPW, EPW)], dstv)
    pltpu.sync_copy(sl_hbm, slv)
    pltpu.sync_copy(sr_hbm, srv)

    def body(i, carry):
        o = i * 16
        s = srcv[pl.ds(o, 16)]
        d = dstv[pl.ds(o, 16)]
        raw = plsc.load_gather(slv, [s]) + plsc.load_gather(srv, [d])
        rawv[pl.ds(o, 16)] = raw
        sigv[pl.ds(o, 16)] = 1.0 / (1.0 + jnp.exp(-raw))
        return carry

    lax.fori_loop(0, EPW // 16, body, 0)
    pltpu.sync_copy(rawv, raw_hbm.at[pl.ds(wid * EPW, EPW)])
    pltpu.sync_copy(sigv, sig_hbm.at[pl.ds(wid * EPW, EPW)])


def _tc_lin0_body(x_ref, w_ref, b_ref, o_ref):
    o_ref[...] = (jnp.dot(x_ref[...], w_ref[...],
                          preferred_element_type=jnp.float32)
                  + b_ref[...][None, :])


def _tc_mid_body(f_ref, c_ref, h0_ref, wl_ref, wr_ref, b_ref,
                 h1p_ref, cnt_ref):
    cnt = jnp.maximum(c_ref[0][:, :1] + c_ref[1][:, :1], 1.0)
    mean = (f_ref[0] + f_ref[1]) / cnt
    h1 = (jnp.dot(mean, wl_ref[...], preferred_element_type=jnp.float32)
          + jnp.dot(h0_ref[...], wr_ref[...],
                    preferred_element_type=jnp.float32)
          + b_ref[...][None, :])
    h1 = jnp.maximum(h1, 0.0)
    h1p_ref[0] = h1[:, :64]
    h1p_ref[1] = h1[:, 64:]
    cnt_ref[...] = cnt


def _tc_out_body(p_ref, cnt_ref, h1p_ref, wl_ref, wr_ref, b_ref,
                 wpl_ref, wpr_ref, bp_ref, sl_ref, sr_ref):
    cnt = cnt_ref[...]
    h2 = (jnp.dot(p_ref[0] / cnt, wl_ref[...][:64],
                  preferred_element_type=jnp.float32)
          + jnp.dot(p_ref[1] / cnt, wl_ref[...][64:],
                    preferred_element_type=jnp.float32)
          + jnp.dot(h1p_ref[0], wr_ref[...][:64],
                    preferred_element_type=jnp.float32)
          + jnp.dot(h1p_ref[1], wr_ref[...][64:],
                    preferred_element_type=jnp.float32)
          + b_ref[...][None, :])
    sl_ref[...] = (jnp.dot(h2, wpl_ref[...],
                           preferred_element_type=jnp.float32)
                   + bp_ref[...][None, :])
    sr_ref[...] = jnp.dot(h2, wpr_ref[...],
                          preferred_element_type=jnp.float32)


def kernel(x, edge_index, edge_attr, W0, b0, Wl1, Wr1, b1,
           Wl2, Wr2, b2, Wp, bp):
    f32 = jnp.float32
    ei3 = edge_index.reshape(2, ECH, CH)

    h0 = pl.pallas_call(
        _tc_lin0_body,
        out_shape=jax.ShapeDtypeStruct((N, 16), f32),
    )(x, W0.T, b0)

    fparts, cparts = _sc_conv1(h0, ei3, jnp.zeros((NP, 16), f32))

    h1p, cntcol = pl.pallas_call(
        _tc_mid_body,
        out_shape=(jax.ShapeDtypeStruct((NCORE, N, 64), f32),
                   jax.ShapeDtypeStruct((N, 1), f32)),
    )(fparts, cparts, h0, Wl1.T, Wr1.T, b1)

    parts2 = _sc_conv2(h1p, ei3, jnp.zeros((NP, 64), f32))

    sl, sr = pl.pallas_call(
        _tc_out_body,
        out_shape=(jax.ShapeDtypeStruct((N, 1), f32),
                   jax.ShapeDtypeStruct((N, 1), f32)),
    )(parts2, cntcol, h1p, Wl2.T, Wr2.T, b2,
      Wp[0, :128].reshape(128, 1), Wp[0, 128:].reshape(128, 1), bp)

    return _sc_head(sl.reshape(N), sr.reshape(N), edge_index)


# speedup vs baseline: 1.6896x; 1.0008x over previous
import functools

import jax
import jax.numpy as jnp
from jax import lax
from jax.experimental import pallas as pl
from jax.experimental.pallas import tpu as pltpu
from jax.experimental.pallas import tpu_sc as plsc

N = 10000
E = 320000
NCORE = 2
NSUB = 16
NW = NCORE * NSUB
CH = 128
ECH = E // CH
NP = 10112
EPW = E // NW
C1B = 78
C1XT = 14
C2B = 156
C2XT = 12
NSLOT = 3
CS2 = 2
ZROWS = NP // NSUB
OROWS = 624
OTAIL = N - OROWS * NSUB

_mesh = plsc.VectorSubcoreMesh(core_axis_name="c", subcore_axis_name="s")
_params = pltpu.CompilerParams(use_tc_tiling_on_sc=False)
_params_head = pltpu.CompilerParams(use_tc_tiling_on_sc=False,
                                    needs_layout_passes=False)


def _copy_out(aggsh, out_hbm, cid, sid):
    pltpu.sync_copy(aggsh.at[pl.ds(sid * OROWS, OROWS)],
                    out_hbm.at[cid, pl.ds(sid * OROWS, OROWS)])

    @pl.when(sid == 0)
    def _():
        pltpu.sync_copy(aggsh.at[pl.ds(OROWS * NSUB, OTAIL)],
                        out_hbm.at[cid, pl.ds(OROWS * NSUB, OTAIL)])


@functools.partial(
    pl.kernel,
    out_type=(jax.ShapeDtypeStruct((NCORE, N, 16), jnp.float32),
              jax.ShapeDtypeStruct((NCORE, N, 16), jnp.float32)),
    mesh=_mesh,
    scratch_types=[
        pltpu.VMEM((C1B + 1, CH), jnp.int32),
        pltpu.VMEM((C1B + 1, CH), jnp.int32),
    ]
    + [pltpu.VMEM((CH, 16), jnp.float32) for _ in range(NSLOT)]
    + [pltpu.VMEM((CH, 16), jnp.float32)]
    + [pltpu.VMEM_SHARED((N, 16), jnp.float32)]
    + [pltpu.VMEM_SHARED((NP, 16), jnp.float32)]
    + [pltpu.VMEM_SHARED((NP, 16), jnp.float32)]
    + [pltpu.SemaphoreType.DMA for _ in range(NSLOT)],
    compiler_params=_params,
)
def _sc_conv1(h0_hbm, ei_hbm, z_hbm, fout_hbm, cout_hbm,
              srcv, dstv, *rest):
    bufs = rest[:NSLOT]
    cbuf = rest[NSLOT]
    h0sh = rest[NSLOT + 1]
    featsh = rest[NSLOT + 2]
    cntsh = rest[NSLOT + 3]
    gsems = rest[NSLOT + 4:]
    cid = lax.axis_index("c")
    sid = lax.axis_index("s")
    base = (cid * (ECH // 2) + sid * C1B
            + jnp.maximum(sid - C1XT, 0).astype(jnp.int32))
    pltpu.sync_copy(ei_hbm.at[0, pl.ds(base, C1B)],
                    srcv.at[pl.ds(0, C1B)])
    pltpu.sync_copy(ei_hbm.at[1, pl.ds(base, C1B)],
                    dstv.at[pl.ds(0, C1B)])

    @pl.when(sid >= C1XT)
    def _():
        pltpu.sync_copy(ei_hbm.at[0, pl.ds(base + C1B, 1)],
                        srcv.at[pl.ds(C1B, 1)])
        pltpu.sync_copy(ei_hbm.at[1, pl.ds(base + C1B, 1)],
                        dstv.at[pl.ds(C1B, 1)])

    pltpu.sync_copy(h0_hbm.at[pl.ds(sid * OROWS, OROWS)],
                    h0sh.at[pl.ds(sid * OROWS, OROWS)])
    pltpu.sync_copy(z_hbm.at[pl.ds(sid * ZROWS, ZROWS)],
                    featsh.at[pl.ds(sid * ZROWS, ZROWS)])
    pltpu.sync_copy(z_hbm.at[pl.ds(sid * ZROWS, ZROWS)],
                    cntsh.at[pl.ds(sid * ZROWS, ZROWS)])

    @pl.when(sid == 0)
    def _():
        pltpu.sync_copy(h0_hbm.at[pl.ds(OROWS * NSUB, OTAIL)],
                        h0sh.at[pl.ds(OROWS * NSUB, OTAIL)])

    ones_col = jnp.where(lax.iota(jnp.int32, 16) == 0, 1.0, 0.0)

    def fill(r, carry):
        cbuf[r] = ones_col
        return carry

    lax.fori_loop(0, CH, fill, 0)
    plsc.subcore_barrier()
    for k in range(NSLOT):
        pltpu.async_copy(h0sh.at[srcv.at[k]], bufs[k], gsems[k])

    def body(t, carry):
        b0 = t * NSLOT
        for k in range(NSLOT):
            j = b0 + k
            pltpu.make_async_copy(h0sh.at[srcv.at[j]],
                                  bufs[k], gsems[k]).wait()
            pltpu.sync_copy(bufs[k], featsh.at[dstv.at[j]], add=True)
            pltpu.sync_copy(cbuf, cntsh.at[dstv.at[j]], add=True)
            nxt = j + NSLOT

            @pl.when(nxt < C1B)
            def _():
                pltpu.async_copy(h0sh.at[srcv.at[nxt]], bufs[k], gsems[k])
        return carry

    lax.fori_loop(0, C1B // NSLOT, body, 0)

    @pl.when(sid >= C1XT)
    def _():
        pltpu.async_copy(h0sh.at[srcv.at[C1B]], bufs[0], gsems[0]).wait()
        pltpu.sync_copy(bufs[0], featsh.at[dstv.at[C1B]], add=True)
        pltpu.sync_copy(cbuf, cntsh.at[dstv.at[C1B]], add=True)

    plsc.subcore_barrier()
    _copy_out(featsh, fout_hbm, cid, sid)
    _copy_out(cntsh, cout_hbm, cid, sid)


@functools.partial(
    pl.kernel,
    out_type=jax.ShapeDtypeStruct((NCORE, N, 64), jnp.float32),
    mesh=_mesh,
    scratch_types=[
        pltpu.VMEM((C2B + 1, CH), jnp.int32),
    ]
    + [pltpu.VMEM((CS2, CH), jnp.int32)]
    + [pltpu.VMEM((CH, 64), jnp.float32) for _ in range(CS2)]
    + [pltpu.VMEM_SHARED((N, 64), jnp.float32)]
    + [pltpu.VMEM_SHARED((NP, 64), jnp.float32)]
    + [pltpu.SemaphoreType.DMA for _ in range(2 * CS2)],
    compiler_params=_params,
)
def _sc_conv2(h1p_hbm, ei_hbm, z_hbm, out_hbm,
              srcv, *rest):
    didx = rest[0]
    bufs = rest[1:CS2 + 1]
    h1sh = rest[CS2 + 1]
    aggsh = rest[CS2 + 2]
    gsems = rest[CS2 + 3:2 * CS2 + 3]
    dsems = rest[2 * CS2 + 3:]
    cid = lax.axis_index("c")
    sid = lax.axis_index("s")
    base = sid * C2B + jnp.maximum(sid - C2XT, 0).astype(jnp.int32)
    pltpu.sync_copy(ei_hbm.at[0, pl.ds(base, C2B)],
                    srcv.at[pl.ds(0, C2B)])

    @pl.when(sid >= C2XT)
    def _():
        pltpu.sync_copy(ei_hbm.at[0, pl.ds(base + C2B, 1)],
                        srcv.at[pl.ds(C2B, 1)])

    pltpu.sync_copy(h1p_hbm.at[cid, pl.ds(sid * OROWS, OROWS)],
                    h1sh.at[pl.ds(sid * OROWS, OROWS)])
    pltpu.sync_copy(z_hbm.at[pl.ds(sid * ZROWS, ZROWS)],
                    aggsh.at[pl.ds(sid * ZROWS, ZROWS)])

    @pl.when(sid == 0)
    def _():
        pltpu.sync_copy(h1p_hbm.at[cid, pl.ds(OROWS * NSUB, OTAIL)],
                        h1sh.at[pl.ds(OROWS * NSUB, OTAIL)])

    plsc.subcore_barrier()
    for k in range(CS2):
        pltpu.async_copy(ei_hbm.at[1, base + k], didx.at[k], dsems[k])
        pltpu.async_copy(h1sh.at[srcv.at[k]], bufs[k], gsems[k])

    def body(t, carry):
        b0 = t * CS2
        for k in range(CS2):
            j = b0 + k
            pltpu.make_async_copy(h1sh.at[srcv.at[j]],
                                  bufs[k], gsems[k]).wait()
            pltpu.make_async_copy(ei_hbm.at[1, base + j],
                                  didx.at[k], dsems[k]).wait()
            pltpu.sync_copy(bufs[k], aggsh.at[didx.at[k]], add=True)
            nxt = j + CS2

            @pl.when(nxt < C2B)
            def _():
                pltpu.async_copy(ei_hbm.at[1, base + nxt], didx.at[k],
                                 dsems[k])
                pltpu.async_copy(h1sh.at[srcv.at[nxt]], bufs[k], gsems[k])
        return carry

    lax.fori_loop(0, C2B // CS2, body, 0)

    @pl.when(sid >= C2XT)
    def _():
        pltpu.async_copy(ei_hbm.at[1, base + C2B], didx.at[0],
                         dsems[0]).wait()
        pltpu.async_copy(h1sh.at[srcv.at[C2B]], bufs[0], gsems[0]).wait()
        pltpu.sync_copy(bufs[0], aggsh.at[didx.at[0]], add=True)

    plsc.subcore_barrier()
    _copy_out(aggsh, out_hbm, cid, sid)


@functools.partial(
    pl.kernel,
    out_type=(jax.ShapeDtypeStruct((E,), jnp.float32),
              jax.ShapeDtypeStruct((E,), jnp.float32)),
    mesh=_mesh,
    scratch_types=[
        pltpu.VMEM((EPW,), jnp.int32),
        pltpu.VMEM((EPW,), jnp.int32),
        pltpu.VMEM((N,), jnp.float32),
        pltpu.VMEM((N,), jnp.float32),
        pltpu.VMEM((EPW,), jnp.float32),
        pltpu.VMEM((EPW,), jnp.float32),
    ],
    compiler_params=_params_head,
)
def _sc_head(sl_hbm, sr_hbm, ei_hbm, raw_hbm, sig_hbm,
             srcv, dstv, slv, srv, rawv, sigv):
    cid = lax.axis_index("c")
    sid = lax.axis_index("s")
    wid = sid * NCORE + cid
    pltpu.sync_copy(ei_hbm.at[0, pl.ds(wid * EPW, EPW)], srcv)
    pltpu.sync_copy(ei_hbm.at[1, pl.ds(wid * EPW, EPW)], dstv)
    pltpu.sync_copy(sl_hbm, slv)
    pltpu.sync_copy(sr_hbm, srv)

    def body(i, carry):
        o = i * 16
        s = srcv[pl.ds(o, 16)]
        d = dstv[pl.ds(o, 16)]
        raw = plsc.load_gather(slv, [s]) + plsc.load_gather(srv, [d])
        rawv[pl.ds(o, 16)] = raw
        sigv[pl.ds(o, 16)] = 1.0 / (1.0 + jnp.exp(-raw))
        return carry

    lax.fori_loop(0, EPW // 16, body, 0)
    pltpu.sync_copy(rawv, raw_hbm.at[pl.ds(wid * EPW, EPW)])
    pltpu.sync_copy(sigv, sig_hbm.at[pl.ds(wid * EPW, EPW)])


def _tc_lin0_body(x_ref, w_ref, b_ref, o_ref):
    o_ref[...] = (jnp.dot(x_ref[...], w_ref[...],
                          preferred_element_type=jnp.float32)
                  + b_ref[...][None, :])


def _tc_mid_body(f_ref, c_ref, h0_ref, wl_ref, wr_ref, b_ref,
                 h1p_ref, cnt_ref):
    cnt = jnp.maximum(c_ref[0][:, :1] + c_ref[1][:, :1], 1.0)
    mean = (f_ref[0] + f_ref[1]) / cnt
    h1 = (jnp.dot(mean, wl_ref[...], preferred_element_type=jnp.float32)
          + jnp.dot(h0_ref[...], wr_ref[...],
                    preferred_element_type=jnp.float32)
          + b_ref[...][None, :])
    h1 = jnp.maximum(h1, 0.0)
    h1p_ref[0] = h1[:, :64]
    h1p_ref[1] = h1[:, 64:]
    cnt_ref[...] = cnt


def _tc_out_body(p_ref, cnt_ref, h1p_ref, wl_ref, wr_ref, b_ref,
                 wpl_ref, wpr_ref, bp_ref, sl_ref, sr_ref):
    cnt = cnt_ref[...]
    h2 = (jnp.dot(p_ref[0] / cnt, wl_ref[...][:64],
                  preferred_element_type=jnp.float32)
          + jnp.dot(p_ref[1] / cnt, wl_ref[...][64:],
                    preferred_element_type=jnp.float32)
          + jnp.dot(h1p_ref[0], wr_ref[...][:64],
                    preferred_element_type=jnp.float32)
          + jnp.dot(h1p_ref[1], wr_ref[...][64:],
                    preferred_element_type=jnp.float32)
          + b_ref[...][None, :])
    sl_ref[...] = jnp.sum(h2 * wpl_ref[...][None, :], axis=1) + bp_ref[...]
    sr_ref[...] = jnp.sum(h2 * wpr_ref[...][None, :], axis=1)


def kernel(x, edge_index, edge_attr, W0, b0, Wl1, Wr1, b1,
           Wl2, Wr2, b2, Wp, bp):
    f32 = jnp.float32
    ei3 = edge_index.reshape(2, ECH, CH)

    blk = N // 10
    h0 = pl.pallas_call(
        _tc_lin0_body,
        grid=(10,),
        in_specs=[pl.BlockSpec((blk, 128), lambda i: (i, 0)),
                  pl.BlockSpec((128, 16), lambda i: (0, 0)),
                  pl.BlockSpec((16,), lambda i: (0,))],
        out_specs=pl.BlockSpec((blk, 16), lambda i: (i, 0)),
        out_shape=jax.ShapeDtypeStruct((N, 16), f32),
    )(x, W0.T, b0)

    fparts, cparts = _sc_conv1(h0, ei3, jnp.zeros((NP, 16), f32))

    h1p, cntcol = pl.pallas_call(
        _tc_mid_body,
        grid=(10,),
        in_specs=[pl.BlockSpec((NCORE, blk, 16), lambda i: (0, i, 0)),
                  pl.BlockSpec((NCORE, blk, 16), lambda i: (0, i, 0)),
                  pl.BlockSpec((blk, 16), lambda i: (i, 0)),
                  pl.BlockSpec((16, 128), lambda i: (0, 0)),
                  pl.BlockSpec((16, 128), lambda i: (0, 0)),
                  pl.BlockSpec((128,), lambda i: (0,))],
        out_specs=(pl.BlockSpec((NCORE, blk, 64), lambda i: (0, i, 0)),
                   pl.BlockSpec((blk, 1), lambda i: (i, 0))),
        out_shape=(jax.ShapeDtypeStruct((NCORE, N, 64), f32),
                   jax.ShapeDtypeStruct((N, 1), f32)),
    )(fparts, cparts, h0, Wl1.T, Wr1.T, b1)

    parts2 = _sc_conv2(h1p, ei3, jnp.zeros((NP, 64), f32))

    sl, sr = pl.pallas_call(
        _tc_out_body,
        out_shape=(jax.ShapeDtypeStruct((N,), f32),
                   jax.ShapeDtypeStruct((N,), f32)),
    )(parts2, cntcol, h1p, Wl2.T, Wr2.T, b2,
      Wp[0, :128], Wp[0, 128:], bp)

    return _sc_head(sl, sr, edge_index)


# speedup vs baseline: 1.7195x; 1.0177x over previous
import functools

import jax
import jax.numpy as jnp
from jax import lax
from jax.experimental import pallas as pl
from jax.experimental.pallas import tpu as pltpu
from jax.experimental.pallas import tpu_sc as plsc

N = 10000
E = 320000
NCORE = 2
NSUB = 16
NW = NCORE * NSUB
CH = 128
ECH = E // CH
NP = 10112
EPW = E // NW
C1B = 78
C1XT = 14
C2B = 156
C2XT = 12
NSLOT = 3
CS2 = 3
ZROWS = NP // NSUB
OROWS = 624
OTAIL = N - OROWS * NSUB

_mesh = plsc.VectorSubcoreMesh(core_axis_name="c", subcore_axis_name="s")
_params = pltpu.CompilerParams(use_tc_tiling_on_sc=False)
_params_head = pltpu.CompilerParams(use_tc_tiling_on_sc=False,
                                    needs_layout_passes=False)


def _copy_out(aggsh, out_hbm, cid, sid):
    pltpu.sync_copy(aggsh.at[pl.ds(sid * OROWS, OROWS)],
                    out_hbm.at[cid, pl.ds(sid * OROWS, OROWS)])

    @pl.when(sid == 0)
    def _():
        pltpu.sync_copy(aggsh.at[pl.ds(OROWS * NSUB, OTAIL)],
                        out_hbm.at[cid, pl.ds(OROWS * NSUB, OTAIL)])


@functools.partial(
    pl.kernel,
    out_type=(jax.ShapeDtypeStruct((NCORE, N, 16), jnp.float32),
              jax.ShapeDtypeStruct((NCORE, N, 16), jnp.float32)),
    mesh=_mesh,
    scratch_types=[
        pltpu.VMEM((C1B + 1, CH), jnp.int32),
        pltpu.VMEM((C1B + 1, CH), jnp.int32),
    ]
    + [pltpu.VMEM((CH, 16), jnp.float32) for _ in range(NSLOT)]
    + [pltpu.VMEM((CH, 16), jnp.float32)]
    + [pltpu.VMEM_SHARED((N, 16), jnp.float32)]
    + [pltpu.VMEM_SHARED((NP, 16), jnp.float32)]
    + [pltpu.VMEM_SHARED((NP, 16), jnp.float32)]
    + [pltpu.SemaphoreType.DMA for _ in range(NSLOT)],
    compiler_params=_params,
)
def _sc_conv1(h0_hbm, ei_hbm, z_hbm, fout_hbm, cout_hbm,
              srcv, dstv, *rest):
    bufs = rest[:NSLOT]
    cbuf = rest[NSLOT]
    h0sh = rest[NSLOT + 1]
    featsh = rest[NSLOT + 2]
    cntsh = rest[NSLOT + 3]
    gsems = rest[NSLOT + 4:]
    cid = lax.axis_index("c")
    sid = lax.axis_index("s")
    base = (cid * (ECH // 2) + sid * C1B
            + jnp.maximum(sid - C1XT, 0).astype(jnp.int32))
    pltpu.sync_copy(ei_hbm.at[0, pl.ds(base, C1B)],
                    srcv.at[pl.ds(0, C1B)])
    pltpu.sync_copy(ei_hbm.at[1, pl.ds(base, C1B)],
                    dstv.at[pl.ds(0, C1B)])

    @pl.when(sid >= C1XT)
    def _():
        pltpu.sync_copy(ei_hbm.at[0, pl.ds(base + C1B, 1)],
                        srcv.at[pl.ds(C1B, 1)])
        pltpu.sync_copy(ei_hbm.at[1, pl.ds(base + C1B, 1)],
                        dstv.at[pl.ds(C1B, 1)])

    pltpu.sync_copy(h0_hbm.at[pl.ds(sid * OROWS, OROWS)],
                    h0sh.at[pl.ds(sid * OROWS, OROWS)])
    pltpu.sync_copy(z_hbm.at[pl.ds(sid * ZROWS, ZROWS)],
                    featsh.at[pl.ds(sid * ZROWS, ZROWS)])
    pltpu.sync_copy(z_hbm.at[pl.ds(sid * ZROWS, ZROWS)],
                    cntsh.at[pl.ds(sid * ZROWS, ZROWS)])

    @pl.when(sid == 0)
    def _():
        pltpu.sync_copy(h0_hbm.at[pl.ds(OROWS * NSUB, OTAIL)],
                        h0sh.at[pl.ds(OROWS * NSUB, OTAIL)])

    ones_col = jnp.where(lax.iota(jnp.int32, 16) == 0, 1.0, 0.0)

    def fill(r, carry):
        cbuf[r] = ones_col
        return carry

    lax.fori_loop(0, CH, fill, 0)
    plsc.subcore_barrier()
    for k in range(NSLOT):
        pltpu.async_copy(h0sh.at[srcv.at[k]], bufs[k], gsems[k])

    def body(t, carry):
        b0 = t * NSLOT
        for k in range(NSLOT):
            j = b0 + k
            pltpu.make_async_copy(h0sh.at[srcv.at[j]],
                                  bufs[k], gsems[k]).wait()
            pltpu.sync_copy(bufs[k], featsh.at[dstv.at[j]], add=True)
            pltpu.sync_copy(cbuf, cntsh.at[dstv.at[j]], add=True)
            nxt = j + NSLOT

            @pl.when(nxt < C1B)
            def _():
                pltpu.async_copy(h0sh.at[srcv.at[nxt]], bufs[k], gsems[k])
        return carry

    lax.fori_loop(0, C1B // NSLOT, body, 0)

    @pl.when(sid >= C1XT)
    def _():
        pltpu.async_copy(h0sh.at[srcv.at[C1B]], bufs[0], gsems[0]).wait()
        pltpu.sync_copy(bufs[0], featsh.at[dstv.at[C1B]], add=True)
        pltpu.sync_copy(cbuf, cntsh.at[dstv.at[C1B]], add=True)

    plsc.subcore_barrier()
    _copy_out(featsh, fout_hbm, cid, sid)
    _copy_out(cntsh, cout_hbm, cid, sid)


@functools.partial(
    pl.kernel,
    out_type=jax.ShapeDtypeStruct((NCORE, N, 64), jnp.float32),
    mesh=_mesh,
    scratch_types=[
        pltpu.VMEM((C2B + 1, CH), jnp.int32),
    ]
    + [pltpu.VMEM((CS2, CH), jnp.int32)]
    + [pltpu.VMEM((CH, 64), jnp.float32) for _ in range(CS2)]
    + [pltpu.VMEM_SHARED((N, 64), jnp.float32)]
    + [pltpu.VMEM_SHARED((NP, 64), jnp.float32)]
    + [pltpu.SemaphoreType.DMA for _ in range(2 * CS2)],
    compiler_params=_params,
)
def _sc_conv2(h1p_hbm, ei_hbm, z_hbm, out_hbm,
              srcv, *rest):
    didx = rest[0]
    bufs = rest[1:CS2 + 1]
    h1sh = rest[CS2 + 1]
    aggsh = rest[CS2 + 2]
    gsems = rest[CS2 + 3:2 * CS2 + 3]
    dsems = rest[2 * CS2 + 3:]
    cid = lax.axis_index("c")
    sid = lax.axis_index("s")
    base = sid * C2B + jnp.maximum(sid - C2XT, 0).astype(jnp.int32)
    pltpu.sync_copy(ei_hbm.at[0, pl.ds(base, C2B)],
                    srcv.at[pl.ds(0, C2B)])

    @pl.when(sid >= C2XT)
    def _():
        pltpu.sync_copy(ei_hbm.at[0, pl.ds(base + C2B, 1)],
                        srcv.at[pl.ds(C2B, 1)])

    pltpu.sync_copy(h1p_hbm.at[cid, pl.ds(sid * OROWS, OROWS)],
                    h1sh.at[pl.ds(sid * OROWS, OROWS)])
    pltpu.sync_copy(z_hbm.at[pl.ds(sid * ZROWS, ZROWS)],
                    aggsh.at[pl.ds(sid * ZROWS, ZROWS)])

    @pl.when(sid == 0)
    def _():
        pltpu.sync_copy(h1p_hbm.at[cid, pl.ds(OROWS * NSUB, OTAIL)],
                        h1sh.at[pl.ds(OROWS * NSUB, OTAIL)])

    plsc.subcore_barrier()
    for k in range(CS2):
        pltpu.async_copy(ei_hbm.at[1, base + k], didx.at[k], dsems[k])
        pltpu.async_copy(h1sh.at[srcv.at[k]], bufs[k], gsems[k])

    def body(t, carry):
        b0 = t * CS2
        for k in range(CS2):
            j = b0 + k
            pltpu.make_async_copy(h1sh.at[srcv.at[j]],
                                  bufs[k], gsems[k]).wait()
            pltpu.make_async_copy(ei_hbm.at[1, base + j],
                                  didx.at[k], dsems[k]).wait()
            pltpu.sync_copy(bufs[k], aggsh.at[didx.at[k]], add=True)
            nxt = j + CS2

            @pl.when(nxt < C2B)
            def _():
                pltpu.async_copy(ei_hbm.at[1, base + nxt], didx.at[k],
                                 dsems[k])
                pltpu.async_copy(h1sh.at[srcv.at[nxt]], bufs[k], gsems[k])
        return carry

    lax.fori_loop(0, C2B // CS2, body, 0)

    @pl.when(sid >= C2XT)
    def _():
        pltpu.async_copy(ei_hbm.at[1, base + C2B], didx.at[0],
                         dsems[0]).wait()
        pltpu.async_copy(h1sh.at[srcv.at[C2B]], bufs[0], gsems[0]).wait()
        pltpu.sync_copy(bufs[0], aggsh.at[didx.at[0]], add=True)

    plsc.subcore_barrier()
    _copy_out(aggsh, out_hbm, cid, sid)


@functools.partial(
    pl.kernel,
    out_type=(jax.ShapeDtypeStruct((E,), jnp.float32),
              jax.ShapeDtypeStruct((E,), jnp.float32)),
    mesh=_mesh,
    scratch_types=[
        pltpu.VMEM((EPW,), jnp.int32),
        pltpu.VMEM((EPW,), jnp.int32),
        pltpu.VMEM((N,), jnp.float32),
        pltpu.VMEM((N,), jnp.float32),
        pltpu.VMEM((EPW,), jnp.float32),
        pltpu.VMEM((EPW,), jnp.float32),
    ],
    compiler_params=_params_head,
)
def _sc_head(sl_hbm, sr_hbm, ei_hbm, raw_hbm, sig_hbm,
             srcv, dstv, slv, srv, rawv, sigv):
    cid = lax.axis_index("c")
    sid = lax.axis_index("s")
    wid = sid * NCORE + cid
    pltpu.sync_copy(ei_hbm.at[0, pl.ds(wid * EPW, EPW)], srcv)
    pltpu.sync_copy(ei_hbm.at[1, pl.ds(wid * EPW, EPW)], dstv)
    pltpu.sync_copy(sl_hbm, slv)
    pltpu.sync_copy(sr_hbm, srv)

    def body(i, carry):
        o = i * 16
        s = srcv[pl.ds(o, 16)]
        d = dstv[pl.ds(o, 16)]
        raw = plsc.load_gather(slv, [s]) + plsc.load_gather(srv, [d])
        rawv[pl.ds(o, 16)] = raw
        sigv[pl.ds(o, 16)] = 1.0 / (1.0 + jnp.exp(-raw))
        return carry

    lax.fori_loop(0, EPW // 16, body, 0)
    pltpu.sync_copy(rawv, raw_hbm.at[pl.ds(wid * EPW, EPW)])
    pltpu.sync_copy(sigv, sig_hbm.at[pl.ds(wid * EPW, EPW)])


def _tc_lin0_body(x_ref, w_ref, b_ref, o_ref):
    o_ref[...] = (jnp.dot(x_ref[...], w_ref[...],
                          preferred_element_type=jnp.float32)
                  + b_ref[...][None, :])


def _tc_mid_body(f_ref, c_ref, h0_ref, wl_ref, wr_ref, b_ref,
                 h1p_ref, cnt_ref):
    cnt = jnp.maximum(c_ref[0][:, :1] + c_ref[1][:, :1], 1.0)
    mean = (f_ref[0] + f_ref[1]) / cnt
    h1 = (jnp.dot(mean, wl_ref[...], preferred_element_type=jnp.float32)
          + jnp.dot(h0_ref[...], wr_ref[...],
                    preferred_element_type=jnp.float32)
          + b_ref[...][None, :])
    h1 = jnp.maximum(h1, 0.0)
    h1p_ref[0] = h1[:, :64]
    h1p_ref[1] = h1[:, 64:]
    cnt_ref[...] = cnt


def _tc_out_body(p_ref, cnt_ref, h1p_ref, wl_ref, wr_ref, b_ref,
                 wpl_ref, wpr_ref, bp_ref, sl_ref, sr_ref):
    cnt = cnt_ref[...]
    h2 = (jnp.dot(p_ref[0] / cnt, wl_ref[...][:64],
                  preferred_element_type=jnp.float32)
          + jnp.dot(p_ref[1] / cnt, wl_ref[...][64:],
                    preferred_element_type=jnp.float32)
          + jnp.dot(h1p_ref[0], wr_ref[...][:64],
                    preferred_element_type=jnp.float32)
          + jnp.dot(h1p_ref[1], wr_ref[...][64:],
                    preferred_element_type=jnp.float32)
          + b_ref[...][None, :])
    sl_ref[...] = jnp.sum(h2 * wpl_ref[...][None, :], axis=1) + bp_ref[...]
    sr_ref[...] = jnp.sum(h2 * wpr_ref[...][None, :], axis=1)


def kernel(x, edge_index, edge_attr, W0, b0, Wl1, Wr1, b1,
           Wl2, Wr2, b2, Wp, bp):
    f32 = jnp.float32
    ei3 = edge_index.reshape(2, ECH, CH)

    h0 = pl.pallas_call(
        _tc_lin0_body,
        out_shape=jax.ShapeDtypeStruct((N, 16), f32),
    )(x, W0.T, b0)

    fparts, cparts = _sc_conv1(h0, ei3, jnp.zeros((NP, 16), f32))

    h1p, cntcol = pl.pallas_call(
        _tc_mid_body,
        out_shape=(jax.ShapeDtypeStruct((NCORE, N, 64), f32),
                   jax.ShapeDtypeStruct((N, 1), f32)),
    )(fparts, cparts, h0, Wl1.T, Wr1.T, b1)

    parts2 = _sc_conv2(h1p, ei3, jnp.zeros((NP, 64), f32))

    sl, sr = pl.pallas_call(
        _tc_out_body,
        out_shape=(jax.ShapeDtypeStruct((N,), f32),
                   jax.ShapeDtypeStruct((N,), f32)),
    )(parts2, cntcol, h1p, Wl2.T, Wr2.T, b2,
      Wp[0, :128], Wp[0, 128:], bp)

    return _sc_head(sl, sr, edge_index)
